# Initial kernel scaffold; baseline (speedup 1.0000x reference)
import functools

import jax
import jax.numpy as jnp
from jax import lax
from jax.experimental import pallas as pl
from jax.experimental.pallas import tpu as pltpu
from jax.experimental.pallas import tpu_sc as plsc

N = 10000
E = 320000
D = 128
NC = 2
NS = 16
NW = NC * NS
EP = E // NW
K = 80
NCHUNK = EP // K
STRIPE = N // NS
ROWS_BLK = 2000
LEAKY = 0.2


def _proj_body(x_ref, w_ref, asrc_ref, adst_ref, h_ref, asd_ref):
    h = jnp.dot(x_ref[:], w_ref[:], preferred_element_type=jnp.float32)
    h_ref[:] = h
    sa = jnp.dot(h, asrc_ref[:].T, preferred_element_type=jnp.float32)
    sd = jnp.dot(h, adst_ref[:].T, preferred_element_type=jnp.float32)
    asd_ref[:] = jnp.concatenate([sa, sd], axis=1)


def _tc_proj(x, w, a_src, a_dst):
    grid = (N // ROWS_BLK,)
    return pl.pallas_call(
        _proj_body,
        grid=grid,
        in_specs=[
            pl.BlockSpec((ROWS_BLK, D), lambda i: (i, 0)),
            pl.BlockSpec((D, D), lambda i: (0, 0)),
            pl.BlockSpec((1, D), lambda i: (0, 0)),
            pl.BlockSpec((1, D), lambda i: (0, 0)),
        ],
        out_specs=[
            pl.BlockSpec((ROWS_BLK, D), lambda i: (i, 0)),
            pl.BlockSpec((ROWS_BLK, 2), lambda i: (i, 0)),
        ],
        out_shape=[
            jax.ShapeDtypeStruct((N, D), jnp.float32),
            jax.ShapeDtypeStruct((N, 2), jnp.float32),
        ],
    )(x, w, a_src.reshape(1, D), a_dst.reshape(1, D))


def _epi_body(s_ref, dp_ref, asd_ref, h_ref, b_ref, w_ref, asrc_ref,
              adst_ref, hout_ref, asd_out_ref, *, relu, final):
    ones = jnp.ones((NW, 1), jnp.float32)
    dsum = lax.dot_general(dp_ref[:], ones, (((0,), (0,)), ((), ())),
                           preferred_element_type=jnp.float32)
    aself = asd_ref[:, 0:1] + asd_ref[:, 1:2]
    exs = jnp.exp(jnp.maximum(aself, LEAKY * aself))
    dsum = dsum + exs
    s = s_ref[0] + s_ref[1] + exs * h_ref[:]
    hin = s / dsum + b_ref[:]
    if relu:
        hin = jnp.maximum(hin, 0.0)
    h2 = jnp.dot(hin, w_ref[:], preferred_element_type=jnp.float32)
    if final:
        hout_ref[:] = h2 + asrc_ref[:]
        asd_out_ref[:] = jnp.zeros((ROWS_BLK, 2), jnp.float32)
    else:
        hout_ref[:] = h2
        sa = jnp.dot(h2, asrc_ref[:].T, preferred_element_type=jnp.float32)
        sd = jnp.dot(h2, adst_ref[:].T, preferred_element_type=jnp.float32)
        asd_out_ref[:] = jnp.concatenate([sa, sd], axis=1)


def _tc_epilogue(s, dpart, asd, h, b, w, a_src, a_dst, relu, final):
    grid = (N // ROWS_BLK,)
    return pl.pallas_call(
        functools.partial(_epi_body, relu=relu, final=final),
        grid=grid,
        in_specs=[
            pl.BlockSpec((2, ROWS_BLK, D), lambda i: (0, i, 0)),
            pl.BlockSpec((NW, ROWS_BLK), lambda i: (0, i)),
            pl.BlockSpec((ROWS_BLK, 2), lambda i: (i, 0)),
            pl.BlockSpec((ROWS_BLK, D), lambda i: (i, 0)),
            pl.BlockSpec((1, D), lambda i: (0, 0)),
            pl.BlockSpec((D, D), lambda i: (0, 0)),
            pl.BlockSpec((1, D), lambda i: (0, 0)),
            pl.BlockSpec((1, D), lambda i: (0, 0)),
        ],
        out_specs=[
            pl.BlockSpec((ROWS_BLK, D), lambda i: (i, 0)),
            pl.BlockSpec((ROWS_BLK, 2), lambda i: (i, 0)),
        ],
        out_shape=[
            jax.ShapeDtypeStruct((N, D), jnp.float32),
            jax.ShapeDtypeStruct((N, 2), jnp.float32),
        ],
    )(s, dpart, asd, h, b.reshape(1, D), w, a_src.reshape(1, D),
      a_dst.reshape(1, D))


def _sc_edge_body(src_hbm, dst_hbm, as_hbm, ad_hbm, h_hbm,
                  s_hbm, dpart_hbm,
                  src_v, dst_v, as_v, ad_v, ex_v, den_v, rows_v, zrow_v,
                  s_sh):
    cid = lax.axis_index("c")
    sid = lax.axis_index("s")
    wid = cid * NS + sid

    pltpu.sync_copy(src_hbm.at[wid], src_v)
    pltpu.sync_copy(dst_hbm.at[wid], dst_v)
    pltpu.sync_copy(as_hbm, as_v)
    pltpu.sync_copy(ad_hbm, ad_v)

    zeros16 = jnp.zeros((16,), jnp.float32)

    def _zden(i, _):
        den_v[pl.ds(i * 16, 16)] = zeros16
        return 0
    lax.fori_loop(0, N // 16, _zden, 0)

    def _zrow(i, _):
        for c in range(D // 16):
            zrow_v[i, pl.ds(c * 16, 16)] = zeros16
        return 0
    lax.fori_loop(0, 125, _zrow, 0)

    def _zs(kk, _):
        pltpu.sync_copy(zrow_v, s_sh.at[pl.ds(sid * STRIPE + kk * 125, 125)])
        return 0
    lax.fori_loop(0, 5, _zs, 0)
    plsc.subcore_barrier()

    def _phase_a(j, _):
        for k in range(K // 16):
            off = j * K + k * 16
            s16 = src_v[pl.ds(off, 16)]
            d16 = dst_v[j, pl.ds(k * 16, 16)]
            av = plsc.load_gather(as_v, [s16])
            dv = plsc.load_gather(ad_v, [d16])
            al = av + dv
            al = jnp.maximum(al, LEAKY * al)
            e = jnp.exp(al)
            ex_v[pl.ds(off, 16)] = e
            plsc.addupdate_scatter(den_v, [d16], e)
        return 0
    lax.fori_loop(0, NCHUNK, _phase_a, 0)
    pltpu.sync_copy(den_v, dpart_hbm.at[wid])

    def _phase_b(j, _):
        pltpu.sync_copy(h_hbm.at[src_v.at[pl.ds(j * K, K)]], rows_v)

        def _scale(r, _):
            gid = j * K + r
            ev = plsc.load_gather(ex_v, [jnp.full((16,), gid, jnp.int32)])
            for c in range(D // 16):
                sl = pl.ds(c * 16, 16)
                rows_v[r, sl] = rows_v[r, sl] * ev
            return 0
        lax.fori_loop(0, K, _scale, 0)

        pltpu.sync_copy(rows_v, s_sh.at[dst_v.at[j]], add=True)
        return 0
    lax.fori_loop(0, NCHUNK, _phase_b, 0)
    plsc.subcore_barrier()

    def _export(kk, _):
        sl = pl.ds(sid * STRIPE + kk * 125, 125)
        pltpu.sync_copy(s_sh.at[sl], s_hbm.at[cid, sl])
        return 0
    lax.fori_loop(0, 5, _export, 0)


def _sc_edge(src2d, dst3d, asv, adv, h):
    mesh = plsc.VectorSubcoreMesh(core_axis_name="c", subcore_axis_name="s")
    f = pl.kernel(
        _sc_edge_body,
        out_type=[
            jax.ShapeDtypeStruct((NC, N, D), jnp.float32),
            jax.ShapeDtypeStruct((NW, N), jnp.float32),
        ],
        mesh=mesh,
        scratch_types=[
            pltpu.VMEM((EP,), jnp.int32),
            pltpu.VMEM((NCHUNK, K), jnp.int32),
            pltpu.VMEM((N,), jnp.float32),
            pltpu.VMEM((N,), jnp.float32),
            pltpu.VMEM((EP,), jnp.float32),
            pltpu.VMEM((N,), jnp.float32),
            pltpu.VMEM((K, D), jnp.float32),
            pltpu.VMEM((125, D), jnp.float32),
            pltpu.VMEM_SHARED((N, D), jnp.float32),
        ],
    )
    return f(src2d, dst3d, asv, adv, h)


def kernel(x, edge_index, edge_attr, W1, a_src1, a_dst1, b1,
           W2, a_src2, a_dst2, b2, Wl, bl):
    del edge_attr
    src2d = edge_index[0].astype(jnp.int32).reshape(NW, EP)
    dst3d = edge_index[1].astype(jnp.int32).reshape(NW, NCHUNK, K)

    h1, asd1 = _tc_proj(x, W1, a_src1, a_dst1)
    s1, dp1 = _sc_edge(src2d, dst3d, asd1[:, 0], asd1[:, 1], h1)
    h2, asd2 = _tc_epilogue(s1, dp1, asd1, h1, b1, W2, a_src2, a_dst2,
                            relu=True, final=False)
    s2, dp2 = _sc_edge(src2d, dst3d, asd2[:, 0], asd2[:, 1], h2)
    out, _ = _tc_epilogue(s2, dp2, asd2, h2, b2, Wl, bl,
                          relu=False, final=True)
    return out


# speedup vs baseline: 25.8836x; 25.8836x over previous
import functools

import jax
import jax.numpy as jnp
from jax import lax
from jax.experimental import pallas as pl
from jax.experimental.pallas import tpu as pltpu
from jax.experimental.pallas import tpu_sc as plsc

N = 10000
E = 320000
D = 128
NC = 2
NS = 16
NW = NC * NS
EP = E // NW
K = 80
NCHUNK = EP // K
ZROWS = 200
ROWS_BLK = 2000
LEAKY = 0.2


def _proj_body(x_ref, w_ref, asrc_ref, adst_ref, h_ref, asd_ref):
    h = jnp.dot(x_ref[:], w_ref[:], preferred_element_type=jnp.float32)
    h_ref[:] = h
    sa = jnp.dot(h, asrc_ref[:].T, preferred_element_type=jnp.float32)
    sd = jnp.dot(h, adst_ref[:].T, preferred_element_type=jnp.float32)
    asd_ref[:] = jnp.concatenate([sa, sd], axis=1)


def _tc_proj(x, w, a_src, a_dst):
    grid = (N // ROWS_BLK,)
    return pl.pallas_call(
        _proj_body,
        grid=grid,
        in_specs=[
            pl.BlockSpec((ROWS_BLK, D), lambda i: (i, 0)),
            pl.BlockSpec((D, D), lambda i: (0, 0)),
            pl.BlockSpec((1, D), lambda i: (0, 0)),
            pl.BlockSpec((1, D), lambda i: (0, 0)),
        ],
        out_specs=[
            pl.BlockSpec((ROWS_BLK, D), lambda i: (i, 0)),
            pl.BlockSpec((ROWS_BLK, 2), lambda i: (i, 0)),
        ],
        out_shape=[
            jax.ShapeDtypeStruct((N, D), jnp.float32),
            jax.ShapeDtypeStruct((N, 2), jnp.float32),
        ],
    )(x, w, a_src.reshape(1, D), a_dst.reshape(1, D))


def _epi_body(s_ref, dp_ref, asd_ref, h_ref, b_ref, w_ref, asrc_ref,
              adst_ref, hout_ref, asd_out_ref, *, relu, final):
    ones = jnp.ones((NW, 1), jnp.float32)
    dsum = lax.dot_general(dp_ref[:], ones, (((1,), (0,)), ((), ())),
                           preferred_element_type=jnp.float32)
    aself = asd_ref[:, 0:1] + asd_ref[:, 1:2]
    exs = jnp.exp(jnp.maximum(aself, LEAKY * aself))
    dsum = dsum + exs
    s = s_ref[0] + s_ref[1] + exs * h_ref[:]
    hin = s / dsum + b_ref[:]
    if relu:
        hin = jnp.maximum(hin, 0.0)
    h2 = jnp.dot(hin, w_ref[:], preferred_element_type=jnp.float32)
    if final:
        hout_ref[:] = h2 + asrc_ref[:]
        asd_out_ref[:] = jnp.zeros((ROWS_BLK, 2), jnp.float32)
    else:
        hout_ref[:] = h2
        sa = jnp.dot(h2, asrc_ref[:].T, preferred_element_type=jnp.float32)
        sd = jnp.dot(h2, adst_ref[:].T, preferred_element_type=jnp.float32)
        asd_out_ref[:] = jnp.concatenate([sa, sd], axis=1)


def _tc_epilogue(s, dpart, asd, h, b, w, a_src, a_dst, relu, final):
    grid = (N // ROWS_BLK,)
    return pl.pallas_call(
        functools.partial(_epi_body, relu=relu, final=final),
        grid=grid,
        in_specs=[
            pl.BlockSpec((2, ROWS_BLK, D), lambda i: (0, i, 0)),
            pl.BlockSpec((ROWS_BLK, NW), lambda i: (i, 0)),
            pl.BlockSpec((ROWS_BLK, 2), lambda i: (i, 0)),
            pl.BlockSpec((ROWS_BLK, D), lambda i: (i, 0)),
            pl.BlockSpec((1, D), lambda i: (0, 0)),
            pl.BlockSpec((D, D), lambda i: (0, 0)),
            pl.BlockSpec((1, D), lambda i: (0, 0)),
            pl.BlockSpec((1, D), lambda i: (0, 0)),
        ],
        out_specs=[
            pl.BlockSpec((ROWS_BLK, D), lambda i: (i, 0)),
            pl.BlockSpec((ROWS_BLK, 2), lambda i: (i, 0)),
        ],
        out_shape=[
            jax.ShapeDtypeStruct((N, D), jnp.float32),
            jax.ShapeDtypeStruct((N, 2), jnp.float32),
        ],
    )(s, dpart.T, asd, h, b.reshape(1, D), w, a_src.reshape(1, D),
      a_dst.reshape(1, D))


def _sc_edge_body(edges_hbm, as_hbm, ad_hbm, h_hbm,
                  s_hbm, dpart_hbm,
                  as_v, ad_v, den_v, rows_v, e_ch, exch_v,
                  s_sh):
    cid = lax.axis_index("c")
    sid = lax.axis_index("s")
    wid = cid * NS + sid

    pltpu.sync_copy(as_hbm, as_v)
    pltpu.sync_copy(ad_hbm, ad_v)

    zeros16 = jnp.zeros((16,), jnp.float32)

    def _zden(i, _):
        den_v[pl.ds(i * 16, 16)] = zeros16
        return 0
    lax.fori_loop(0, N // 16, _zden, 0)

    def _zrow(i, _):
        for c in range(D // 16):
            rows_v[i, pl.ds(c * 16, 16)] = zeros16
        return 0
    lax.fori_loop(0, K, _zrow, 0)

    c0 = sid
    nz = N // K
    def _zs_strided(i, _):
        c = c0 + i * NS
        @pl.when(c < nz)
        def _():
            pltpu.sync_copy(rows_v, s_sh.at[pl.ds(c * K, K)])
        return 0
    lax.fori_loop(0, (nz + NS - 1) // NS, _zs_strided, 0)
    plsc.subcore_barrier()

    def _chunk(j, _):
        pltpu.sync_copy(edges_hbm.at[wid, j], e_ch)

        for k in range(K // 16):
            sl = pl.ds(k * 16, 16)
            s16 = e_ch[0, sl]
            d16 = e_ch[1, sl]
            av = plsc.load_gather(as_v, [s16])
            dv = plsc.load_gather(ad_v, [d16])
            al = av + dv
            al = jnp.maximum(al, LEAKY * al)
            e = jnp.exp(al)
            exch_v[sl] = e
            plsc.addupdate_scatter(den_v, [d16], e)

        pltpu.sync_copy(h_hbm.at[e_ch.at[0]], rows_v)

        def _scale(r, _):
            ev = plsc.load_gather(exch_v, [jnp.full((16,), r, jnp.int32)])
            for c in range(D // 16):
                sl = pl.ds(c * 16, 16)
                rows_v[r, sl] = rows_v[r, sl] * ev
            return 0
        lax.fori_loop(0, K, _scale, 0)

        pltpu.sync_copy(rows_v, s_sh.at[e_ch.at[1]], add=True)
        return 0
    lax.fori_loop(0, NCHUNK, _chunk, 0)

    pltpu.sync_copy(den_v, dpart_hbm.at[wid])
    plsc.subcore_barrier()

    def _export(i, _):
        c = c0 + i * NS
        @pl.when(c < nz)
        def _():
            sl = pl.ds(c * K, K)
            pltpu.sync_copy(s_sh.at[sl], s_hbm.at[cid, sl])
        return 0
    lax.fori_loop(0, (nz + NS - 1) // NS, _export, 0)


def _sc_edge(edges4d, asv, adv, h):
    mesh = plsc.VectorSubcoreMesh(core_axis_name="c", subcore_axis_name="s")
    f = pl.kernel(
        _sc_edge_body,
        out_type=[
            jax.ShapeDtypeStruct((NC, N, D), jnp.float32),
            jax.ShapeDtypeStruct((NW, N), jnp.float32),
        ],
        mesh=mesh,
        compiler_params=pltpu.CompilerParams(needs_layout_passes=False),
        scratch_types=[
            pltpu.VMEM((N,), jnp.float32),
            pltpu.VMEM((N,), jnp.float32),
            pltpu.VMEM((N,), jnp.float32),
            pltpu.VMEM((K, D), jnp.float32),
            pltpu.VMEM((2, K), jnp.int32),
            pltpu.VMEM((K,), jnp.float32),
            pltpu.VMEM_SHARED((N, D), jnp.float32),
        ],
    )
    return f(edges4d, asv, adv, h)


def kernel(x, edge_index, edge_attr, W1, a_src1, a_dst1, b1,
           W2, a_src2, a_dst2, b2, Wl, bl):
    del edge_attr
    ei = edge_index.astype(jnp.int32).reshape(2, NW, NCHUNK, K)
    edges4d = jnp.stack([ei[0], ei[1]], axis=2)

    h1, asd1 = _tc_proj(x, W1, a_src1, a_dst1)
    s1, dp1 = _sc_edge(edges4d, asd1[:, 0], asd1[:, 1], h1)
    h2, asd2 = _tc_epilogue(s1, dp1, asd1, h1, b1, W2, a_src2, a_dst2,
                            relu=True, final=False)
    s2, dp2 = _sc_edge(edges4d, asd2[:, 0], asd2[:, 1], h2)
    out, _ = _tc_epilogue(s2, dp2, asd2, h2, b2, Wl, bl, bl,
                          relu=False, final=True)
    return out


# speedup vs baseline: 47.0046x; 1.8160x over previous
import functools

import jax
import jax.numpy as jnp
from jax import lax
from jax.experimental import pallas as pl
from jax.experimental.pallas import tpu as pltpu
from jax.experimental.pallas import tpu_sc as plsc

N = 10000
E = 320000
D = 128
NC = 2
NS = 16
NW = NC * NS
EP = E // NW
K = 80
NCHUNK = EP // K
ZROWS = 200
ROWS_BLK = 2000
LEAKY = 0.2


def _proj_body(x_ref, w_ref, asrc_ref, adst_ref, h_ref, asd_ref):
    h = jnp.dot(x_ref[:], w_ref[:], preferred_element_type=jnp.float32)
    h_ref[:] = h
    sa = jnp.dot(h, asrc_ref[:].T, preferred_element_type=jnp.float32)
    sd = jnp.dot(h, adst_ref[:].T, preferred_element_type=jnp.float32)
    asd_ref[:] = jnp.concatenate([sa, sd], axis=1)


def _tc_proj(x, w, a_src, a_dst):
    grid = (N // ROWS_BLK,)
    return pl.pallas_call(
        _proj_body,
        grid=grid,
        in_specs=[
            pl.BlockSpec((ROWS_BLK, D), lambda i: (i, 0)),
            pl.BlockSpec((D, D), lambda i: (0, 0)),
            pl.BlockSpec((1, D), lambda i: (0, 0)),
            pl.BlockSpec((1, D), lambda i: (0, 0)),
        ],
        out_specs=[
            pl.BlockSpec((ROWS_BLK, D), lambda i: (i, 0)),
            pl.BlockSpec((ROWS_BLK, 2), lambda i: (i, 0)),
        ],
        out_shape=[
            jax.ShapeDtypeStruct((N, D), jnp.float32),
            jax.ShapeDtypeStruct((N, 2), jnp.float32),
        ],
    )(x, w, a_src.reshape(1, D), a_dst.reshape(1, D))


def _epi_body(s_ref, dp_ref, asd_ref, h_ref, b_ref, w_ref, asrc_ref,
              adst_ref, hout_ref, asd_out_ref, *, relu, final):
    dsum = dp_ref[:, 0:1] + dp_ref[:, 1:2]
    aself = asd_ref[:, 0:1] + asd_ref[:, 1:2]
    exs = jnp.exp(jnp.maximum(aself, LEAKY * aself))
    dsum = dsum + exs
    s = s_ref[0] + s_ref[1] + exs * h_ref[:]
    hin = s / dsum + b_ref[:]
    if relu:
        hin = jnp.maximum(hin, 0.0)
    h2 = jnp.dot(hin, w_ref[:], preferred_element_type=jnp.float32)
    if final:
        hout_ref[:] = h2 + asrc_ref[:]
        asd_out_ref[:] = jnp.zeros((ROWS_BLK, 2), jnp.float32)
    else:
        hout_ref[:] = h2
        sa = jnp.dot(h2, asrc_ref[:].T, preferred_element_type=jnp.float32)
        sd = jnp.dot(h2, adst_ref[:].T, preferred_element_type=jnp.float32)
        asd_out_ref[:] = jnp.concatenate([sa, sd], axis=1)


def _tc_epilogue(s, dpart, asd, h, b, w, a_src, a_dst, relu, final):
    grid = (N // ROWS_BLK,)
    return pl.pallas_call(
        functools.partial(_epi_body, relu=relu, final=final),
        grid=grid,
        in_specs=[
            pl.BlockSpec((2, ROWS_BLK, D), lambda i: (0, i, 0)),
            pl.BlockSpec((ROWS_BLK, NC), lambda i: (i, 0)),
            pl.BlockSpec((ROWS_BLK, 2), lambda i: (i, 0)),
            pl.BlockSpec((ROWS_BLK, D), lambda i: (i, 0)),
            pl.BlockSpec((1, D), lambda i: (0, 0)),
            pl.BlockSpec((D, D), lambda i: (0, 0)),
            pl.BlockSpec((1, D), lambda i: (0, 0)),
            pl.BlockSpec((1, D), lambda i: (0, 0)),
        ],
        out_specs=[
            pl.BlockSpec((ROWS_BLK, D), lambda i: (i, 0)),
            pl.BlockSpec((ROWS_BLK, 2), lambda i: (i, 0)),
        ],
        out_shape=[
            jax.ShapeDtypeStruct((N, D), jnp.float32),
            jax.ShapeDtypeStruct((N, 2), jnp.float32),
        ],
    )(s, dpart.T, asd, h, b.reshape(1, D), w, a_src.reshape(1, D),
      a_dst.reshape(1, D))


def _sc_edge_body(edges_hbm, as_hbm, ad_hbm, h_hbm,
                  s_hbm, dpart_hbm,
                  as_v, ad_v, rows0, rows1, e0, e1, e2, e3,
                  exch0, exch1, dscat0, dscat1, zden,
                  es0, es1, es2, es3, gs0, gs1, ss0, ss1, ds0, ds1,
                  s_sh, den_sh):
    cid = lax.axis_index("c")
    sid = lax.axis_index("s")
    wid = cid * NS + sid
    rows = [rows0, rows1]
    e_ch = [e0, e1, e2, e3]
    exch = [exch0, exch1]
    dscat = [dscat0, dscat1]
    e_sem = [es0, es1, es2, es3]
    g_sem = [gs0, gs1]
    s_sem = [ss0, ss1]
    d_sem = [ds0, ds1]

    for c in range(4):
        pltpu.async_copy(edges_hbm.at[wid, c], e_ch[c], e_sem[c])
    pltpu.sync_copy(as_hbm, as_v)
    pltpu.sync_copy(ad_hbm, ad_v)

    zeros16 = jnp.zeros((16,), jnp.float32)
    for k in range(K // 16):
        zden[pl.ds(k * 16, 16)] = zeros16

    def _zrow(i, _):
        for c in range(D // 16):
            rows0[i, pl.ds(c * 16, 16)] = zeros16
        return 0
    lax.fori_loop(0, K, _zrow, 0)

    nz = N // K
    def _zs_strided(i, _):
        c = sid + i * NS
        @pl.when(c < nz)
        def _():
            pltpu.sync_copy(rows0, s_sh.at[pl.ds(c * K, K)])
            pltpu.sync_copy(zden, den_sh.at[pl.ds(c * K, K)])
        return 0
    lax.fori_loop(0, (nz + NS - 1) // NS, _zs_strided, 0)
    plsc.subcore_barrier()

    def _macro(t, _):
        for u in range(4):
            j = t * 4 + u
            b = u % 2
            eb = u
            eb_prev = (u + 3) % 4

            @pl.when(j <= NCHUNK - 1)
            def _stage_a():
                pltpu.make_async_copy(
                    edges_hbm.at[wid, j], e_ch[eb], e_sem[eb]).wait()

                @pl.when(j >= 2)
                def _():
                    pltpu.make_async_copy(
                        exch[b], den_sh.at[dscat[b]], d_sem[b]).wait()
                    pltpu.make_async_copy(
                        rows[b], s_sh.at[dscat[b]], s_sem[b]).wait()

                for k in range(K // 16):
                    sl = pl.ds(k * 16, 16)
                    s16 = e_ch[eb][0, sl]
                    d16 = e_ch[eb][1, sl]
                    av = plsc.load_gather(as_v, [s16])
                    dv = plsc.load_gather(ad_v, [d16])
                    al = av + dv
                    al = jnp.maximum(al, LEAKY * al)
                    exch[b][sl] = jnp.exp(al)
                    dscat[b][sl] = d16
                pltpu.async_copy(exch[b], den_sh.at[dscat[b]], d_sem[b],
                                 add=True)
                pltpu.async_copy(h_hbm.at[e_ch[eb].at[0]], rows[b],
                                 g_sem[b])

            @pl.when((j >= 1) & (j <= NCHUNK))
            def _stage_b():
                b2 = 1 - b
                pltpu.make_async_copy(
                    h_hbm.at[e_ch[eb_prev].at[0]], rows[b2],
                    g_sem[b2]).wait()

                def _scale(r, _):
                    ev = plsc.load_gather(
                        exch[b2], [jnp.full((16,), r, jnp.int32)])
                    for c in range(D // 16):
                        sl = pl.ds(c * 16, 16)
                        rows[b2][r, sl] = rows[b2][r, sl] * ev
                    return 0
                lax.fori_loop(0, K, _scale, 0)

                pltpu.async_copy(rows[b2], s_sh.at[dscat[b2]], s_sem[b2],
                                 add=True)

                @pl.when(j + 3 <= NCHUNK - 1)
                def _():
                    pltpu.async_copy(edges_hbm.at[wid, j + 3],
                                     e_ch[eb_prev], e_sem[eb_prev])
        return 0
    lax.fori_loop(0, (NCHUNK + 4) // 4, _macro, 0)

    for b in range(2):
        pltpu.make_async_copy(exch[b], den_sh.at[dscat[b]], d_sem[b]).wait()
        pltpu.make_async_copy(rows[b], s_sh.at[dscat[b]], s_sem[b]).wait()
    plsc.subcore_barrier()

    @pl.when(sid == 0)
    def _():
        pltpu.sync_copy(den_sh, dpart_hbm.at[cid])

    def _export(i, _):
        c = sid + i * NS
        @pl.when(c < nz)
        def _():
            sl = pl.ds(c * K, K)
            pltpu.sync_copy(s_sh.at[sl], s_hbm.at[cid, sl])
        return 0
    lax.fori_loop(0, (nz + NS - 1) // NS, _export, 0)


def _sc_edge(edges4d, asv, adv, h):
    mesh = plsc.VectorSubcoreMesh(core_axis_name="c", subcore_axis_name="s")
    f = pl.kernel(
        _sc_edge_body,
        out_type=[
            jax.ShapeDtypeStruct((NC, N, D), jnp.float32),
            jax.ShapeDtypeStruct((NC, N), jnp.float32),
        ],
        mesh=mesh,
        compiler_params=pltpu.CompilerParams(needs_layout_passes=False),
        scratch_types=[
            pltpu.VMEM((N,), jnp.float32),
            pltpu.VMEM((N,), jnp.float32),
            pltpu.VMEM((K, D), jnp.float32),
            pltpu.VMEM((K, D), jnp.float32),
            pltpu.VMEM((2, K), jnp.int32),
            pltpu.VMEM((2, K), jnp.int32),
            pltpu.VMEM((2, K), jnp.int32),
            pltpu.VMEM((2, K), jnp.int32),
            pltpu.VMEM((K,), jnp.float32),
            pltpu.VMEM((K,), jnp.float32),
            pltpu.VMEM((K,), jnp.int32),
            pltpu.VMEM((K,), jnp.int32),
            pltpu.VMEM((K,), jnp.float32),
            pltpu.SemaphoreType.DMA,
            pltpu.SemaphoreType.DMA,
            pltpu.SemaphoreType.DMA,
            pltpu.SemaphoreType.DMA,
            pltpu.SemaphoreType.DMA,
            pltpu.SemaphoreType.DMA,
            pltpu.SemaphoreType.DMA,
            pltpu.SemaphoreType.DMA,
            pltpu.SemaphoreType.DMA,
            pltpu.SemaphoreType.DMA,
            pltpu.VMEM_SHARED((N, D), jnp.float32),
            pltpu.VMEM_SHARED((N,), jnp.float32),
        ],
    )
    return f(edges4d, asv, adv, h)


def kernel(x, edge_index, edge_attr, W1, a_src1, a_dst1, b1,
           W2, a_src2, a_dst2, b2, Wl, bl):
    del edge_attr
    ei = edge_index.astype(jnp.int32).reshape(2, NW, NCHUNK, K)
    edges4d = jnp.stack([ei[0], ei[1]], axis=2)

    h1, asd1 = _tc_proj(x, W1, a_src1, a_dst1)
    s1, dp1 = _sc_edge(edges4d, asd1[:, 0], asd1[:, 1], h1)
    h2, asd2 = _tc_epilogue(s1, dp1, asd1, h1, b1, W2, a_src2, a_dst2,
                            relu=True, final=False)
    s2, dp2 = _sc_edge(edges4d, asd2[:, 0], asd2[:, 1], h2)
    out, _ = _tc_epilogue(s2, dp2, asd2, h2, b2, Wl, bl, bl,
                          relu=False, final=True)
    return out


# speedup vs baseline: 55.4748x; 1.1802x over previous
import functools

import jax
import jax.numpy as jnp
from jax import lax
from jax.experimental import pallas as pl
from jax.experimental.pallas import tpu as pltpu
from jax.experimental.pallas import tpu_sc as plsc

N = 10000
E = 320000
D = 128
NC = 2
NS = 16
NW = NC * NS
EP = E // NW
K = 80
NCHUNK = EP // K
ZROWS = 200
ROWS_BLK = 2000
LEAKY = 0.2


def _proj_body(x_ref, w_ref, asrc_ref, adst_ref, h_ref, asd_ref):
    h = jnp.dot(x_ref[:], w_ref[:], preferred_element_type=jnp.float32)
    h_ref[:] = h
    sa = jnp.dot(h, asrc_ref[:].T, preferred_element_type=jnp.float32)
    sd = jnp.dot(h, adst_ref[:].T, preferred_element_type=jnp.float32)
    asd_ref[:] = jnp.concatenate([sa, sd], axis=1)


def _tc_proj(x, w, a_src, a_dst):
    grid = (N // ROWS_BLK,)
    return pl.pallas_call(
        _proj_body,
        grid=grid,
        in_specs=[
            pl.BlockSpec((ROWS_BLK, D), lambda i: (i, 0)),
            pl.BlockSpec((D, D), lambda i: (0, 0)),
            pl.BlockSpec((1, D), lambda i: (0, 0)),
            pl.BlockSpec((1, D), lambda i: (0, 0)),
        ],
        out_specs=[
            pl.BlockSpec((ROWS_BLK, D), lambda i: (i, 0)),
            pl.BlockSpec((ROWS_BLK, 2), lambda i: (i, 0)),
        ],
        out_shape=[
            jax.ShapeDtypeStruct((N, D), jnp.float32),
            jax.ShapeDtypeStruct((N, 2), jnp.float32),
        ],
    )(x, w, a_src.reshape(1, D), a_dst.reshape(1, D))


def _epi_body(s_ref, dp_ref, asd_ref, h_ref, b_ref, w_ref, asrc_ref,
              adst_ref, hout_ref, asd_out_ref, *, relu, final):
    dsum = dp_ref[:, 0:1] + dp_ref[:, 1:2]
    aself = asd_ref[:, 0:1] + asd_ref[:, 1:2]
    exs = jnp.exp(jnp.maximum(aself, LEAKY * aself))
    dsum = dsum + exs
    s = s_ref[0] + s_ref[1] + exs * h_ref[:]
    hin = s / dsum + b_ref[:]
    if relu:
        hin = jnp.maximum(hin, 0.0)
    h2 = jnp.dot(hin, w_ref[:], preferred_element_type=jnp.float32)
    if final:
        hout_ref[:] = h2 + asrc_ref[:]
        asd_out_ref[:] = jnp.zeros((ROWS_BLK, 2), jnp.float32)
    else:
        hout_ref[:] = h2
        sa = jnp.dot(h2, asrc_ref[:].T, preferred_element_type=jnp.float32)
        sd = jnp.dot(h2, adst_ref[:].T, preferred_element_type=jnp.float32)
        asd_out_ref[:] = jnp.concatenate([sa, sd], axis=1)


def _tc_epilogue(s, dpart, asd, h, b, w, a_src, a_dst, relu, final):
    grid = (N // ROWS_BLK,)
    return pl.pallas_call(
        functools.partial(_epi_body, relu=relu, final=final),
        grid=grid,
        in_specs=[
            pl.BlockSpec((2, ROWS_BLK, D), lambda i: (0, i, 0)),
            pl.BlockSpec((ROWS_BLK, NC), lambda i: (i, 0)),
            pl.BlockSpec((ROWS_BLK, 2), lambda i: (i, 0)),
            pl.BlockSpec((ROWS_BLK, D), lambda i: (i, 0)),
            pl.BlockSpec((1, D), lambda i: (0, 0)),
            pl.BlockSpec((D, D), lambda i: (0, 0)),
            pl.BlockSpec((1, D), lambda i: (0, 0)),
            pl.BlockSpec((1, D), lambda i: (0, 0)),
        ],
        out_specs=[
            pl.BlockSpec((ROWS_BLK, D), lambda i: (i, 0)),
            pl.BlockSpec((ROWS_BLK, 2), lambda i: (i, 0)),
        ],
        out_shape=[
            jax.ShapeDtypeStruct((N, D), jnp.float32),
            jax.ShapeDtypeStruct((N, 2), jnp.float32),
        ],
    )(s, dpart.T, asd, h, b.reshape(1, D), w, a_src.reshape(1, D),
      a_dst.reshape(1, D))


def _sc_edge_body(edges_hbm, as_hbm, ad_hbm, h_hbm,
                  s_hbm, dpart_hbm,
                  as_v, ad_v, rows0, rows1, e0, e1, e2, e3,
                  exch0, exch1, dscat0, dscat1, zden,
                  es0, es1, es2, es3, gs0, gs1, ss0, ss1, ds0, ds1,
                  s_sh, den_sh):
    cid = lax.axis_index("c")
    sid = lax.axis_index("s")
    wid = cid * NS + sid
    rows = [rows0, rows1]
    e_ch = [e0, e1, e2, e3]
    exch = [exch0, exch1]
    dscat = [dscat0, dscat1]
    e_sem = [es0, es1, es2, es3]
    g_sem = [gs0, gs1]
    s_sem = [ss0, ss1]
    d_sem = [ds0, ds1]

    for c in range(4):
        pltpu.async_copy(edges_hbm.at[wid, c], e_ch[c], e_sem[c])
    pltpu.sync_copy(as_hbm, as_v)
    pltpu.sync_copy(ad_hbm, ad_v)

    zeros16 = jnp.zeros((16,), jnp.float32)
    for k in range(K // 16):
        zden[pl.ds(k * 16, 16)] = zeros16

    def _zrow(i, _):
        for c in range(D // 16):
            rows0[i, pl.ds(c * 16, 16)] = zeros16
        return 0
    lax.fori_loop(0, K, _zrow, 0)

    nz = N // K
    def _zs_strided(i, _):
        c = sid + i * NS
        @pl.when(c < nz)
        def _():
            pltpu.sync_copy(rows0, s_sh.at[pl.ds(c * K, K)])
            pltpu.sync_copy(zden, den_sh.at[pl.ds(c * K, K)])
        return 0
    lax.fori_loop(0, (nz + NS - 1) // NS, _zs_strided, 0)
    plsc.subcore_barrier()

    def _macro(t, _):
        for u in range(4):
            j = t * 4 + u
            b = u % 2
            eb = u
            eb_prev = (u + 3) % 4

            @pl.when(j <= NCHUNK - 1)
            def _stage_a():
                pltpu.make_async_copy(
                    edges_hbm.at[wid, j], e_ch[eb], e_sem[eb]).wait()

                @pl.when(j >= 2)
                def _():
                    pltpu.make_async_copy(
                        exch[b], den_sh.at[dscat[b]], d_sem[b]).wait()
                    pltpu.make_async_copy(
                        rows[b], s_sh.at[dscat[b]], s_sem[b]).wait()

                for k in range(K // 16):
                    sl = pl.ds(k * 16, 16)
                    s16 = e_ch[eb][0, sl]
                    d16 = e_ch[eb][1, sl]
                    av = plsc.load_gather(as_v, [s16])
                    dv = plsc.load_gather(ad_v, [d16])
                    al = av + dv
                    al = jnp.maximum(al, LEAKY * al)
                    exch[b][sl] = jnp.exp(al)
                    dscat[b][sl] = d16
                pltpu.async_copy(exch[b], den_sh.at[dscat[b]], d_sem[b],
                                 add=True)
                pltpu.async_copy(h_hbm.at[e_ch[eb].at[0]], rows[b],
                                 g_sem[b])

            @pl.when((j >= 1) & (j <= NCHUNK))
            def _stage_b():
                b2 = 1 - b
                pltpu.make_async_copy(
                    h_hbm.at[e_ch[eb_prev].at[0]], rows[b2],
                    g_sem[b2]).wait()

                @plsc.parallel_loop(0, K, unroll=4)
                def _scale(r):
                    ev = plsc.load_gather(
                        exch[b2], [jnp.full((16,), r, jnp.int32)])
                    for c in range(D // 16):
                        sl = pl.ds(c * 16, 16)
                        rows[b2][r, sl] = rows[b2][r, sl] * ev

                pltpu.async_copy(rows[b2], s_sh.at[dscat[b2]], s_sem[b2],
                                 add=True)

                @pl.when(j + 3 <= NCHUNK - 1)
                def _():
                    pltpu.async_copy(edges_hbm.at[wid, j + 3],
                                     e_ch[eb_prev], e_sem[eb_prev])
        return 0
    lax.fori_loop(0, (NCHUNK + 4) // 4, _macro, 0)

    for b in range(2):
        pltpu.make_async_copy(exch[b], den_sh.at[dscat[b]], d_sem[b]).wait()
        pltpu.make_async_copy(rows[b], s_sh.at[dscat[b]], s_sem[b]).wait()
    plsc.subcore_barrier()

    @pl.when(sid == 0)
    def _():
        pltpu.sync_copy(den_sh, dpart_hbm.at[cid])

    def _export(i, _):
        c = sid + i * NS
        @pl.when(c < nz)
        def _():
            sl = pl.ds(c * K, K)
            pltpu.sync_copy(s_sh.at[sl], s_hbm.at[cid, sl])
        return 0
    lax.fori_loop(0, (nz + NS - 1) // NS, _export, 0)


def _sc_edge(edges4d, asv, adv, h):
    mesh = plsc.VectorSubcoreMesh(core_axis_name="c", subcore_axis_name="s")
    f = pl.kernel(
        _sc_edge_body,
        out_type=[
            jax.ShapeDtypeStruct((NC, N, D), jnp.float32),
            jax.ShapeDtypeStruct((NC, N), jnp.float32),
        ],
        mesh=mesh,
        compiler_params=pltpu.CompilerParams(needs_layout_passes=False),
        scratch_types=[
            pltpu.VMEM((N,), jnp.float32),
            pltpu.VMEM((N,), jnp.float32),
            pltpu.VMEM((K, D), jnp.float32),
            pltpu.VMEM((K, D), jnp.float32),
            pltpu.VMEM((2, K), jnp.int32),
            pltpu.VMEM((2, K), jnp.int32),
            pltpu.VMEM((2, K), jnp.int32),
            pltpu.VMEM((2, K), jnp.int32),
            pltpu.VMEM((K,), jnp.float32),
            pltpu.VMEM((K,), jnp.float32),
            pltpu.VMEM((K,), jnp.int32),
            pltpu.VMEM((K,), jnp.int32),
            pltpu.VMEM((K,), jnp.float32),
            pltpu.SemaphoreType.DMA,
            pltpu.SemaphoreType.DMA,
            pltpu.SemaphoreType.DMA,
            pltpu.SemaphoreType.DMA,
            pltpu.SemaphoreType.DMA,
            pltpu.SemaphoreType.DMA,
            pltpu.SemaphoreType.DMA,
            pltpu.SemaphoreType.DMA,
            pltpu.SemaphoreType.DMA,
            pltpu.SemaphoreType.DMA,
            pltpu.VMEM_SHARED((N, D), jnp.float32),
            pltpu.VMEM_SHARED((N,), jnp.float32),
        ],
    )
    return f(edges4d, asv, adv, h)


def kernel(x, edge_index, edge_attr, W1, a_src1, a_dst1, b1,
           W2, a_src2, a_dst2, b2, Wl, bl):
    del edge_attr
    ei = edge_index.astype(jnp.int32).reshape(2, NW, NCHUNK, K)
    edges4d = jnp.stack([ei[0], ei[1]], axis=2)

    h1, asd1 = _tc_proj(x, W1, a_src1, a_dst1)
    s1, dp1 = _sc_edge(edges4d, asd1[:, 0], asd1[:, 1], h1)
    h2, asd2 = _tc_epilogue(s1, dp1, asd1, h1, b1, W2, a_src2, a_dst2,
                            relu=True, final=False)
    s2, dp2 = _sc_edge(edges4d, asd2[:, 0], asd2[:, 1], h2)
    out, _ = _tc_epilogue(s2, dp2, asd2, h2, b2, Wl, bl, bl,
                          relu=False, final=True)
    return out
